# Initial kernel scaffold; baseline (speedup 1.0000x reference)
import jax
import jax.numpy as jnp
from jax.experimental import pallas as pl


def kernel(z, u, table):
    raise NotImplementedError("write your pallas kernel here")


# speedup vs baseline: 1.1130x; 1.1130x over previous
import functools

import jax
import jax.numpy as jnp
from jax import lax
from jax.experimental import pallas as pl
from jax.experimental.pallas import tpu as pltpu
from jax.experimental.pallas import tpu_sc as plsc

NC = 2
NS = 16
NW = NC * NS
IDX_W = 128
IDX_ROWS = 8
SUPER = IDX_ROWS * IDX_W
HALF = SUPER // 2
PAD_D = 128


@functools.partial(jax.jit, static_argnums=(3,))
def _embed_add(table_p, zidx, u_p, n):
    b_per_w = n // NW
    steps_per_w = b_per_w // SUPER
    mesh = plsc.VectorSubcoreMesh(core_axis_name="core", subcore_axis_name="sub")

    @functools.partial(
        pl.kernel,
        out_type=jax.ShapeDtypeStruct((n, PAD_D), jnp.float32),
        mesh=mesh,
        scratch_types=[
            pltpu.VMEM((IDX_ROWS, IDX_W), jnp.int32),
            pltpu.VMEM((HALF, PAD_D), jnp.float32),
            pltpu.SemaphoreType.DMA,
            pltpu.SemaphoreType.DMA,
        ],
    )
    def k(table_hbm, z_hbm, u_hbm, out_hbm, idx_v, rows_v, sem_u, sem_g):
        wid = lax.axis_index("sub") * NC + lax.axis_index("core")

        @pl.loop(0, steps_per_w)
        def _(ci):
            base = pl.multiple_of(wid * b_per_w + ci * SUPER, SUPER)
            ib = pl.multiple_of(base // IDX_W, IDX_ROWS)
            pltpu.sync_copy(z_hbm.at[pl.ds(ib, IDX_ROWS)], idx_v)
            for r in range(IDX_ROWS):
                for c0 in range(0, IDX_W, 16):
                    idx_v[r, pl.ds(c0, 16)] = idx_v[r, pl.ds(c0, 16)] + 1
            for h in range(2):
                hbase = base + h * HALF
                pltpu.async_copy(
                    u_hbm.at[pl.ds(hbase, HALF)], rows_v, sem_u
                ).wait()
                gs = [
                    pltpu.async_copy(
                        table_hbm.at[idx_v.at[h * (IDX_ROWS // 2) + r]],
                        rows_v.at[pl.ds(r * IDX_W, IDX_W)],
                        sem_g,
                        add=True,
                    )
                    for r in range(IDX_ROWS // 2)
                ]
                for g in gs:
                    g.wait()
                pltpu.sync_copy(rows_v, out_hbm.at[pl.ds(hbase, HALF)])

    return k(table_p, zidx, u_p)


def kernel(z, u, table):
    b, l = z.shape
    d = table.shape[1]
    n = b * l
    table_p = jnp.pad(table, ((0, 0), (0, PAD_D - d)))
    zidx = z.astype(jnp.int32).reshape(n // IDX_W, IDX_W)
    u_p = jnp.pad(u.reshape(n, d), ((0, 0), (0, PAD_D - d)))
    vp = _embed_add(table_p, zidx, u_p, n)
    return (z, vp[:, :d].reshape(b, l, d))


# speedup vs baseline: 1.3997x; 1.2576x over previous
import functools

import jax
import jax.numpy as jnp
from jax import lax
from jax.experimental import pallas as pl
from jax.experimental.pallas import tpu as pltpu
from jax.experimental.pallas import tpu_sc as plsc

NC = 2
NS = 16
NW = NC * NS
PAD_D = 128
WB = 8
PAD_ROWS = 8000


def _pad_body(t_ref, o_ref):
    o_ref[:, 0:64] = t_ref[...]


def _pad_table(table):
    v, d = table.shape
    return pl.pallas_call(
        _pad_body,
        grid=(v // PAD_ROWS,),
        in_specs=[pl.BlockSpec((PAD_ROWS, d), lambda i: (i, 0))],
        out_specs=pl.BlockSpec((PAD_ROWS, PAD_D), lambda i: (i, 0)),
        out_shape=jax.ShapeDtypeStruct((v, PAD_D), jnp.float32),
    )(table)


@jax.jit
def _embed_add(table_p, z, u):
    b, l = z.shape
    d = u.shape[-1]
    b_per_w = b // NW
    chunks_per_w = b_per_w // WB
    mesh = plsc.VectorSubcoreMesh(core_axis_name="core", subcore_axis_name="sub")

    @functools.partial(
        pl.kernel,
        out_type=jax.ShapeDtypeStruct((b, l, d), jnp.float32),
        mesh=mesh,
        scratch_types=[
            pltpu.VMEM((WB, l), jnp.int32),
            pltpu.VMEM((WB, l, d), jnp.float32),
            pltpu.VMEM((WB * l, PAD_D), jnp.float32),
            pltpu.SemaphoreType.DMA,
            pltpu.SemaphoreType.DMA,
        ],
    )
    def k(table_hbm, z_hbm, u_hbm, out_hbm, idx_v, u_v, rows_v, sem_u, sem_g):
        wid = lax.axis_index("sub") * NC + lax.axis_index("core")

        @pl.loop(0, chunks_per_w)
        def _(ci):
            b0 = pl.multiple_of(wid * b_per_w + ci * WB, WB)
            u_cp = pltpu.async_copy(u_hbm.at[pl.ds(b0, WB)], u_v, sem_u)
            pltpu.sync_copy(z_hbm.at[pl.ds(b0, WB)], idx_v)
            tail_inc = jnp.where(lax.iota(jnp.int32, 16) >= 14, 1, 0)
            for r in range(WB):
                for c0 in (0, 16, 32):
                    idx_v[r, pl.ds(c0, 16)] = idx_v[r, pl.ds(c0, 16)] + 1
                idx_v[r, pl.ds(34, 16)] = idx_v[r, pl.ds(34, 16)] + tail_inc
            gs = [
                pltpu.async_copy(
                    table_hbm.at[idx_v.at[r]],
                    rows_v.at[pl.ds(r * l, l)],
                    sem_g,
                )
                for r in range(WB)
            ]
            u_cp.wait()
            for g in gs:
                g.wait()

            @pl.loop(0, l)
            def _(li):
                for r in range(WB):
                    for c0 in (0, 16, 32, 48):
                        u_v[r, li, pl.ds(c0, 16)] = (
                            u_v[r, li, pl.ds(c0, 16)]
                            + rows_v[r * l + li, pl.ds(c0, 16)]
                        )

            pltpu.sync_copy(u_v, out_hbm.at[pl.ds(b0, WB)])

    return k(table_p, z, u)


def kernel(z, u, table):
    table_p = _pad_table(table)
    v = _embed_add(table_p, z.astype(jnp.int32), u)
    return (z, v)


# speedup vs baseline: 1.5086x; 1.0778x over previous
import functools

import jax
import jax.numpy as jnp
from jax import lax
from jax.experimental import pallas as pl
from jax.experimental.pallas import tpu as pltpu
from jax.experimental.pallas import tpu_sc as plsc

NC = 2
NS = 16
NW = NC * NS
PAD_D = 128
WB = 4
PAIR_ROWS = 8
PAD_ROWS = 8000


def _pad_body(t_ref, o_ref):
    o_ref[:, 0:64] = t_ref[...]


def _pad_table(table):
    v, d = table.shape
    return pl.pallas_call(
        _pad_body,
        grid=(v // PAD_ROWS,),
        in_specs=[pl.BlockSpec((PAD_ROWS, d), lambda i: (i, 0))],
        out_specs=pl.BlockSpec((PAD_ROWS, PAD_D), lambda i: (i, 0)),
        out_shape=jax.ShapeDtypeStruct((v, PAD_D), jnp.float32),
    )(table)


@jax.jit
def _embed_add(table_p, z, u):
    b, l = z.shape
    d = u.shape[-1]
    b_per_w = b // NW
    n_chunks = b_per_w // WB
    n_macro = n_chunks // 4
    mesh = plsc.VectorSubcoreMesh(core_axis_name="core", subcore_axis_name="sub")

    @functools.partial(
        pl.kernel,
        out_type=jax.ShapeDtypeStruct((b, l, d), jnp.float32),
        mesh=mesh,
        scratch_types=[
            pltpu.VMEM((PAIR_ROWS, l), jnp.int32),
            pltpu.VMEM((PAIR_ROWS, l), jnp.int32),
            pltpu.VMEM((WB, l, d), jnp.float32),
            pltpu.VMEM((WB, l, d), jnp.float32),
            pltpu.VMEM((WB * l, PAD_D), jnp.float32),
            pltpu.VMEM((WB * l, PAD_D), jnp.float32),
        ] + [pltpu.SemaphoreType.DMA] * 8,
    )
    def k(table_hbm, z_hbm, u_hbm, out_hbm,
          idx0, idx1, ub0, ub1, rb0, rb1,
          su0, su1, sg0, sg1, so0, so1, si0, si1):
        idxs = (idx0, idx1)
        us = (ub0, ub1)
        rs = (rb0, rb1)
        sus = (su0, su1)
        sgs = (sg0, sg1)
        sos = (so0, so1)
        sis = (si0, si1)
        wid = lax.axis_index("sub") * NC + lax.axis_index("core")
        w0 = wid * b_per_w

        tail_inc = jnp.where(lax.iota(jnp.int32, 16) >= 14, 1, 0)

        def inc(jb):
            ib = idxs[jb]
            for r in range(PAIR_ROWS):
                for c0 in (0, 16, 32):
                    ib[r, pl.ds(c0, 16)] = ib[r, pl.ds(c0, 16)] + 1
                ib[r, pl.ds(34, 16)] = ib[r, pl.ds(34, 16)] + tail_inc

        def idx_copy(pj, jb):
            z0 = pl.multiple_of(w0 + pj * PAIR_ROWS, PAIR_ROWS)
            return pltpu.make_async_copy(
                z_hbm.at[pl.ds(z0, PAIR_ROWS)], idxs[jb], sis[jb])

        def u_copy(ci, p):
            b0 = w0 + ci * WB
            return pltpu.make_async_copy(
                u_hbm.at[pl.ds(b0, WB)], us[p], sus[p])

        def g_copy(r, p, jb, q):
            return pltpu.make_async_copy(
                table_hbm.at[idxs[jb].at[q * WB + r]],
                rs[p].at[pl.ds(r * l, l)],
                sgs[p])

        def o_copy(ci, p):
            b0 = w0 + ci * WB
            return pltpu.make_async_copy(
                us[p], out_hbm.at[pl.ds(b0, WB)], sos[p])

        def start_a(ci, p, jb, q):
            u_copy(ci, p).start()
            for r in range(WB):
                g_copy(r, p, jb, q).start()

        def do_b(ci, p):
            u_copy(ci, p).wait()
            for r in range(WB):
                g_copy(r, p, 0, 0).wait()

            @pl.loop(0, l)
            def _(li):
                for r in range(WB):
                    for c0 in (0, 16, 32, 48):
                        us[p][r, li, pl.ds(c0, 16)] = (
                            us[p][r, li, pl.ds(c0, 16)]
                            + rs[p][r * l + li, pl.ds(c0, 16)]
                        )

            o_copy(ci, p).start()

        c = idx_copy(0, 0)
        c.start()
        c.wait()
        inc(0)
        idx_copy(1, 1).start()
        start_a(0, 0, 0, 0)

        @pl.loop(0, n_macro)
        def _(mi):
            c0 = mi * 4

            @pl.when(mi > 0)
            def _():
                o_copy(0, 1).wait()

            start_a(c0 + 1, 1, 0, 1)
            do_b(c0, 0)
            idx_copy(0, 1).wait()
            inc(1)
            o_copy(0, 0).wait()
            start_a(c0 + 2, 0, 1, 0)
            do_b(c0 + 1, 1)

            @pl.when(mi < n_macro - 1)
            def _():
                idx_copy(2 * mi + 2, 0).start()

            o_copy(0, 1).wait()
            start_a(c0 + 3, 1, 1, 1)
            do_b(c0 + 2, 0)

            @pl.when(mi < n_macro - 1)
            def _():
                idx_copy(0, 0).wait()
                inc(0)
                o_copy(0, 0).wait()
                start_a(c0 + 4, 0, 0, 0)

            do_b(c0 + 3, 1)

            @pl.when(mi < n_macro - 1)
            def _():
                idx_copy(2 * mi + 3, 1).start()

        o_copy(0, 0).wait()
        o_copy(0, 1).wait()

    return k(table_p, z, u)


def kernel(z, u, table):
    table_p = _pad_table(table)
    v = _embed_add(table_p, z.astype(jnp.int32), u)
    return (z, v)


# speedup vs baseline: 1.7071x; 1.1316x over previous
import functools

import jax
import jax.numpy as jnp
from jax import lax
from jax.experimental import pallas as pl
from jax.experimental.pallas import tpu as pltpu
from jax.experimental.pallas import tpu_sc as plsc

NC = 2
NS = 16
NW = NC * NS
PAD_D = 128
WB = 4
PAIR_ROWS = 8
PAD_COLS = 2048


def _pad_body(tt_ref, o_ref):
    o_ref[:, 0:64] = tt_ref[...].T


def _pad_table(table):
    v, d = table.shape
    return pl.pallas_call(
        _pad_body,
        grid=(pl.cdiv(v, PAD_COLS),),
        in_specs=[pl.BlockSpec((d, PAD_COLS), lambda i: (0, i))],
        out_specs=pl.BlockSpec((PAD_COLS, PAD_D), lambda i: (i, 0)),
        out_shape=jax.ShapeDtypeStruct((v, PAD_D), jnp.float32),
    )(table.T)


@jax.jit
def _embed_add(table_p, z, u):
    b, l = z.shape
    d = u.shape[-1]
    b_per_w = b // NW
    n_chunks = b_per_w // WB
    n_macro = n_chunks // 4
    mesh = plsc.VectorSubcoreMesh(core_axis_name="core", subcore_axis_name="sub")

    @functools.partial(
        pl.kernel,
        out_type=jax.ShapeDtypeStruct((b, l, d), jnp.float32),
        mesh=mesh,
        scratch_types=[
            pltpu.VMEM((PAIR_ROWS, l), jnp.int32),
            pltpu.VMEM((PAIR_ROWS, l), jnp.int32),
            pltpu.VMEM((WB, l, d), jnp.float32),
            pltpu.VMEM((WB, l, d), jnp.float32),
            pltpu.VMEM((WB * l, PAD_D), jnp.float32),
            pltpu.VMEM((WB * l, PAD_D), jnp.float32),
        ] + [pltpu.SemaphoreType.DMA] * 8,
    )
    def k(table_hbm, z_hbm, u_hbm, out_hbm,
          idx0, idx1, ub0, ub1, rb0, rb1,
          su0, su1, sg0, sg1, so0, so1, si0, si1):
        idxs = (idx0, idx1)
        us = (ub0, ub1)
        rs = (rb0, rb1)
        sus = (su0, su1)
        sgs = (sg0, sg1)
        sos = (so0, so1)
        sis = (si0, si1)
        wid = lax.axis_index("sub") * NC + lax.axis_index("core")
        w0 = wid * b_per_w

        tail_inc = jnp.where(lax.iota(jnp.int32, 16) >= 14, 1, 0)

        def inc(jb):
            ib = idxs[jb]
            for r in range(PAIR_ROWS):
                for c0 in (0, 16, 32):
                    ib[r, pl.ds(c0, 16)] = ib[r, pl.ds(c0, 16)] + 1
                ib[r, pl.ds(34, 16)] = ib[r, pl.ds(34, 16)] + tail_inc

        def idx_copy(pj, jb):
            z0 = pl.multiple_of(w0 + pj * PAIR_ROWS, PAIR_ROWS)
            return pltpu.make_async_copy(
                z_hbm.at[pl.ds(z0, PAIR_ROWS)], idxs[jb], sis[jb])

        def u_copy(ci, p):
            b0 = w0 + ci * WB
            return pltpu.make_async_copy(
                u_hbm.at[pl.ds(b0, WB)], us[p], sus[p])

        def g_copy(r, p, jb, q):
            return pltpu.make_async_copy(
                table_hbm.at[idxs[jb].at[q * WB + r]],
                rs[p].at[pl.ds(r * l, l)],
                sgs[p])

        def o_copy(ci, p):
            b0 = w0 + ci * WB
            return pltpu.make_async_copy(
                us[p], out_hbm.at[pl.ds(b0, WB)], sos[p])

        def start_a(ci, p, jb, q):
            u_copy(ci, p).start()
            for r in range(WB):
                g_copy(r, p, jb, q).start()

        def do_b(ci, p):
            u_copy(ci, p).wait()
            for r in range(WB):
                g_copy(r, p, 0, 0).wait()

            @pl.loop(0, l)
            def _(li):
                for r in range(WB):
                    for c0 in (0, 16, 32, 48):
                        us[p][r, li, pl.ds(c0, 16)] = (
                            us[p][r, li, pl.ds(c0, 16)]
                            + rs[p][r * l + li, pl.ds(c0, 16)]
                        )

            o_copy(ci, p).start()

        c = idx_copy(0, 0)
        c.start()
        c.wait()
        inc(0)
        idx_copy(1, 1).start()
        start_a(0, 0, 0, 0)

        @pl.loop(0, n_macro)
        def _(mi):
            c0 = mi * 4

            @pl.when(mi > 0)
            def _():
                o_copy(0, 1).wait()

            start_a(c0 + 1, 1, 0, 1)
            do_b(c0, 0)
            idx_copy(0, 1).wait()
            inc(1)
            o_copy(0, 0).wait()
            start_a(c0 + 2, 0, 1, 0)
            do_b(c0 + 1, 1)

            @pl.when(mi < n_macro - 1)
            def _():
                idx_copy(2 * mi + 2, 0).start()

            o_copy(0, 1).wait()
            start_a(c0 + 3, 1, 1, 1)
            do_b(c0 + 2, 0)

            @pl.when(mi < n_macro - 1)
            def _():
                idx_copy(0, 0).wait()
                inc(0)
                o_copy(0, 0).wait()
                start_a(c0 + 4, 0, 0, 0)

            do_b(c0 + 3, 1)

            @pl.when(mi < n_macro - 1)
            def _():
                idx_copy(2 * mi + 3, 1).start()

        o_copy(0, 0).wait()
        o_copy(0, 1).wait()

    return k(table_p, z, u)


def kernel(z, u, table):
    table_p = _pad_table(table)
    v = _embed_add(table_p, z.astype(jnp.int32), u)
    return (z, v)
